# Initial kernel scaffold; baseline (speedup 1.0000x reference)
import jax
import jax.numpy as jnp
from jax.experimental import pallas as pl


def kernel(q, qr, pid, q_emb, q_emb_diff, qr_emb, qr_emb_diff, diff_emb):
    raise NotImplementedError("write your pallas kernel here")


# speedup vs baseline: 16.5826x; 16.5826x over previous
import functools

import jax
import jax.numpy as jnp
from jax import lax
from jax.experimental import pallas as pl
from jax.experimental.pallas import tpu as pltpu
from jax.experimental.pallas import tpu_sc as plsc

_L2 = 1e-05
_LANES = 16
_NC, _NS = 2, 16
_NW = _NC * _NS
_CHUNK = 128


def _combine_sc(qf, qrf, pidf, q_emb, q_emb_diff, qr_emb, qr_emb_diff, d16):
    n = qf.shape[0]
    d = q_emb.shape[1]
    per_w = n // _NW
    steps = per_w // _CHUNK
    nchunk = d // _LANES
    mesh = plsc.VectorSubcoreMesh(core_axis_name="c", subcore_axis_name="s")

    @functools.partial(
        pl.kernel,
        mesh=mesh,
        compiler_params=pltpu.CompilerParams(use_tc_tiling_on_sc=False),
        out_type=[
            jax.ShapeDtypeStruct((n, d), jnp.float32),
            jax.ShapeDtypeStruct((n, d), jnp.float32),
            jax.ShapeDtypeStruct((_NW, _LANES), jnp.float32),
        ],
        scratch_types=[
            pltpu.VMEM((_CHUNK,), jnp.int32),
            pltpu.VMEM((_CHUNK,), jnp.int32),
            pltpu.VMEM((_CHUNK,), jnp.int32),
            pltpu.VMEM((_CHUNK, d), jnp.float32),
            pltpu.VMEM((_CHUNK, d), jnp.float32),
            pltpu.VMEM((_CHUNK, d), jnp.float32),
            pltpu.VMEM((_CHUNK, d), jnp.float32),
            pltpu.VMEM((_CHUNK, _LANES), jnp.float32),
            pltpu.VMEM((_LANES,), jnp.float32),
            pltpu.SemaphoreType.DMA,
        ],
    )
    def k(qf_hbm, qrf_hbm, pidf_hbm, qe_t, qed_t, qre_t, qred_t, d16_t,
          x_hbm, y_hbm, part_hbm,
          iq_v, iqr_v, ipid_v, qe_v, qed_v, qre_v, qred_v, d_v, acc_v, sem):
        wid = lax.axis_index("s") * _NC + lax.axis_index("c")
        acc_v[...] = jnp.zeros((_LANES,), jnp.float32)

        @pl.loop(0, steps)
        def _(step):
            base = wid * per_w + step * _CHUNK
            pltpu.sync_copy(qf_hbm.at[pl.ds(base, _CHUNK)], iq_v)
            pltpu.sync_copy(qrf_hbm.at[pl.ds(base, _CHUNK)], iqr_v)
            pltpu.sync_copy(pidf_hbm.at[pl.ds(base, _CHUNK)], ipid_v)
            c1 = pltpu.async_copy(qe_t.at[iq_v], qe_v, sem)
            c2 = pltpu.async_copy(qed_t.at[iq_v], qed_v, sem)
            c3 = pltpu.async_copy(qre_t.at[iqr_v], qre_v, sem)
            c4 = pltpu.async_copy(qred_t.at[iqr_v], qred_v, sem)
            c5 = pltpu.async_copy(d16_t.at[ipid_v], d_v, sem)
            c1.wait()
            c2.wait()
            c3.wait()
            c4.wait()
            c5.wait()

            @pl.loop(0, _CHUNK)
            def _(i):
                dv = d_v[i, pl.ds(0, _LANES)]
                acc_v[...] += dv * dv
                for cc in range(nchunk):
                    sl = pl.ds(cc * _LANES, _LANES)
                    qe_v[i, sl] = qe_v[i, sl] + dv * qed_v[i, sl]
                    qre_v[i, sl] = qre_v[i, sl] + dv * qred_v[i, sl]

            pltpu.sync_copy(qe_v, x_hbm.at[pl.ds(base, _CHUNK)])
            pltpu.sync_copy(qre_v, y_hbm.at[pl.ds(base, _CHUNK)])

        pltpu.sync_copy(acc_v, part_hbm.at[wid])

    return k(qf, qrf, pidf, q_emb, q_emb_diff, qr_emb, qr_emb_diff, d16)


def _loss_tc(partials):
    def body(p_ref, o_ref):
        o_ref[0, 0] = jnp.sum(p_ref[...]) * jnp.float32(_L2 / _LANES)

    return pl.pallas_call(
        body,
        out_shape=jax.ShapeDtypeStruct((1, 1), jnp.float32),
        out_specs=pl.BlockSpec(memory_space=pltpu.SMEM),
    )(partials)


def kernel(q, qr, pid, q_emb, q_emb_diff, qr_emb, qr_emb_diff, diff_emb):
    b, l = q.shape
    d = q_emb.shape[1]
    qf = q.reshape(-1).astype(jnp.int32)
    qrf = qr.reshape(-1).astype(jnp.int32)
    pidf = pid.reshape(-1).astype(jnp.int32)
    d16 = jnp.broadcast_to(diff_emb, (diff_emb.shape[0], _LANES))
    x, y, parts = _combine_sc(
        qf, qrf, pidf, q_emb, q_emb_diff, qr_emb, qr_emb_diff, d16)
    loss = _loss_tc(parts)[0, 0]
    return x.reshape(b, l, d), y.reshape(b, l, d), loss


# speedup vs baseline: 19.6697x; 1.1862x over previous
import functools

import jax
import jax.numpy as jnp
from jax import lax
from jax.experimental import pallas as pl
from jax.experimental.pallas import tpu as pltpu
from jax.experimental.pallas import tpu_sc as plsc

_L2 = 1e-05
_LANES = 16
_NC, _NS = 2, 16
_NW = _NC * _NS
_CHUNK = 128


def _combine_sc(qf, qrf, pidf, q_emb, q_emb_diff, qr_emb, qr_emb_diff, d16):
    n = qf.shape[0]
    d = q_emb.shape[1]
    per_w = n // _NW
    steps = per_w // _CHUNK
    nchunk = d // _LANES
    assert steps % 2 == 0
    mesh = plsc.VectorSubcoreMesh(core_axis_name="c", subcore_axis_name="s")

    @functools.partial(
        pl.kernel,
        mesh=mesh,
        compiler_params=pltpu.CompilerParams(use_tc_tiling_on_sc=False),
        out_type=[
            jax.ShapeDtypeStruct((n, d), jnp.float32),
            jax.ShapeDtypeStruct((n, d), jnp.float32),
            jax.ShapeDtypeStruct((_NW, _LANES), jnp.float32),
        ],
        scratch_types=[
            pltpu.VMEM((2, _CHUNK), jnp.int32),
            pltpu.VMEM((2, _CHUNK), jnp.int32),
            pltpu.VMEM((2, _CHUNK), jnp.int32),
            pltpu.VMEM((2, _CHUNK, d), jnp.float32),
            pltpu.VMEM((2, _CHUNK, d), jnp.float32),
            pltpu.VMEM((2, _CHUNK, d), jnp.float32),
            pltpu.VMEM((2, _CHUNK, d), jnp.float32),
            pltpu.VMEM((2, _CHUNK, _LANES), jnp.float32),
            pltpu.VMEM((_LANES,), jnp.float32),
            pltpu.SemaphoreType.DMA,
            pltpu.SemaphoreType.DMA,
            pltpu.SemaphoreType.DMA,
            pltpu.SemaphoreType.DMA,
        ],
    )
    def k(qf_hbm, qrf_hbm, pidf_hbm, qe_t, qed_t, qre_t, qred_t, d16_t,
          x_hbm, y_hbm, part_hbm,
          iq_v, iqr_v, ipid_v, qe_v, qed_v, qre_v, qred_v, d_v, acc_v,
          gsem0, gsem1, ssem0, ssem1):
        wid = lax.axis_index("s") * _NC + lax.axis_index("c")
        gsem = (gsem0, gsem1)
        ssem = (ssem0, ssem1)
        acc_v[...] = jnp.zeros((_LANES,), jnp.float32)

        def fire(b, step):
            base = wid * per_w + step * _CHUNK
            pltpu.sync_copy(qf_hbm.at[pl.ds(base, _CHUNK)], iq_v.at[b])
            pltpu.sync_copy(qrf_hbm.at[pl.ds(base, _CHUNK)], iqr_v.at[b])
            pltpu.sync_copy(pidf_hbm.at[pl.ds(base, _CHUNK)], ipid_v.at[b])
            pltpu.async_copy(qe_t.at[iq_v.at[b]], qe_v.at[b], gsem[b])
            pltpu.async_copy(qed_t.at[iq_v.at[b]], qed_v.at[b], gsem[b])
            pltpu.async_copy(qre_t.at[iqr_v.at[b]], qre_v.at[b], gsem[b])
            pltpu.async_copy(qred_t.at[iqr_v.at[b]], qred_v.at[b], gsem[b])
            pltpu.async_copy(d16_t.at[ipid_v.at[b]], d_v.at[b], gsem[b])

        def wait_g(b):
            pltpu.make_async_copy(qe_t.at[iq_v.at[b]], qe_v.at[b], gsem[b]).wait()
            pltpu.make_async_copy(qed_t.at[iq_v.at[b]], qed_v.at[b], gsem[b]).wait()
            pltpu.make_async_copy(qre_t.at[iqr_v.at[b]], qre_v.at[b], gsem[b]).wait()
            pltpu.make_async_copy(qred_t.at[iqr_v.at[b]], qred_v.at[b], gsem[b]).wait()
            pltpu.make_async_copy(d16_t.at[ipid_v.at[b]], d_v.at[b], gsem[b]).wait()

        def compute(b):
            @pl.loop(0, _CHUNK)
            def _(i):
                dv = d_v[b, i, pl.ds(0, _LANES)]
                acc_v[...] += dv * dv
                for cc in range(nchunk):
                    sl = pl.ds(cc * _LANES, _LANES)
                    qe_v[b, i, sl] = qe_v[b, i, sl] + dv * qed_v[b, i, sl]
                    qre_v[b, i, sl] = qre_v[b, i, sl] + dv * qred_v[b, i, sl]

        def store(b, step):
            base = wid * per_w + step * _CHUNK
            pltpu.async_copy(qe_v.at[b], x_hbm.at[pl.ds(base, _CHUNK)], ssem[b])
            pltpu.async_copy(qre_v.at[b], y_hbm.at[pl.ds(base, _CHUNK)], ssem[b])

        def wait_s(b):
            pltpu.make_async_copy(qe_v.at[b], x_hbm.at[pl.ds(0, _CHUNK)], ssem[b]).wait()
            pltpu.make_async_copy(qre_v.at[b], y_hbm.at[pl.ds(0, _CHUNK)], ssem[b]).wait()

        fire(0, 0)

        @pl.loop(0, steps // 2)
        def _(p):
            s0 = 2 * p
            s1 = s0 + 1
            wait_g(0)
            fire(1, s1)
            compute(0)
            store(0, s0)
            wait_g(1)
            wait_s(0)

            @pl.when(s1 + 1 < steps)
            def _():
                fire(0, s1 + 1)

            compute(1)
            store(1, s1)
            wait_s(1)

        pltpu.sync_copy(acc_v, part_hbm.at[wid])

    return k(qf, qrf, pidf, q_emb, q_emb_diff, qr_emb, qr_emb_diff, d16)


def _loss_tc(partials):
    def body(p_ref, o_ref):
        o_ref[0, 0] = jnp.sum(p_ref[...]) * jnp.float32(_L2 / _LANES)

    return pl.pallas_call(
        body,
        out_shape=jax.ShapeDtypeStruct((1, 1), jnp.float32),
        out_specs=pl.BlockSpec(memory_space=pltpu.SMEM),
    )(partials)


def kernel(q, qr, pid, q_emb, q_emb_diff, qr_emb, qr_emb_diff, diff_emb):
    b, l = q.shape
    d = q_emb.shape[1]
    qf = q.reshape(-1).astype(jnp.int32)
    qrf = qr.reshape(-1).astype(jnp.int32)
    pidf = pid.reshape(-1).astype(jnp.int32)
    d16 = jnp.broadcast_to(diff_emb, (diff_emb.shape[0], _LANES))
    x, y, parts = _combine_sc(
        qf, qrf, pidf, q_emb, q_emb_diff, qr_emb, qr_emb_diff, d16)
    loss = _loss_tc(parts)[0, 0]
    return x.reshape(b, l, d), y.reshape(b, l, d), loss


# speedup vs baseline: 23.6189x; 1.2008x over previous
import functools

import jax
import jax.numpy as jnp
from jax import lax
from jax.experimental import pallas as pl
from jax.experimental.pallas import tpu as pltpu
from jax.experimental.pallas import tpu_sc as plsc

_L2 = 1e-05
_LANES = 16
_NC, _NS = 2, 16
_NW = _NC * _NS
_CHUNK = 128


def _combine_sc(qf, qrf, pidf, qq, rr, dflat):
    n = qf.shape[0]
    d2 = qq.shape[1]
    d = d2 // 2
    per_w = n // _NW
    steps = per_w // _CHUNK
    nchunk = d // _LANES
    assert steps % 2 == 0
    mesh = plsc.VectorSubcoreMesh(core_axis_name="c", subcore_axis_name="s")

    @functools.partial(
        pl.kernel,
        mesh=mesh,
        compiler_params=pltpu.CompilerParams(needs_layout_passes=False),
        out_type=[
            jax.ShapeDtypeStruct((n, d), jnp.float32),
            jax.ShapeDtypeStruct((n, d), jnp.float32),
            jax.ShapeDtypeStruct((_NW, _LANES), jnp.float32),
        ],
        scratch_types=[
            pltpu.VMEM((2, _CHUNK), jnp.int32),
            pltpu.VMEM((2, _CHUNK), jnp.int32),
            pltpu.VMEM((2, _CHUNK), jnp.int32),
            pltpu.VMEM((2, _CHUNK, d2), jnp.float32),
            pltpu.VMEM((2, _CHUNK, d2), jnp.float32),
            pltpu.VMEM((2, _CHUNK), jnp.float32),
            pltpu.VMEM((_CHUNK, d), jnp.float32),
            pltpu.VMEM((_CHUNK, d), jnp.float32),
            pltpu.VMEM((_LANES,), jnp.float32),
            pltpu.SemaphoreType.DMA,
            pltpu.SemaphoreType.DMA,
            pltpu.SemaphoreType.DMA,
        ],
    )
    def k(qf_hbm, qrf_hbm, pidf_hbm, qq_t, rr_t, df_t,
          x_hbm, y_hbm, part_hbm,
          iq_v, iqr_v, ipid_v, tq_v, tr_v, dc_v, xo_v, yo_v, acc_v,
          gsem0, gsem1, ssem):
        wid = lax.axis_index("s") * _NC + lax.axis_index("c")
        gsem = (gsem0, gsem1)
        acc_v[...] = jnp.zeros((_LANES,), jnp.float32)

        def fire(b, step):
            base = wid * per_w + step * _CHUNK
            pltpu.sync_copy(qf_hbm.at[pl.ds(base, _CHUNK)], iq_v.at[b])
            pltpu.sync_copy(qrf_hbm.at[pl.ds(base, _CHUNK)], iqr_v.at[b])
            pltpu.sync_copy(pidf_hbm.at[pl.ds(base, _CHUNK)], ipid_v.at[b])
            pltpu.async_copy(qq_t.at[iq_v.at[b]], tq_v.at[b], gsem[b])
            pltpu.async_copy(rr_t.at[iqr_v.at[b]], tr_v.at[b], gsem[b])
            pltpu.async_copy(df_t.at[ipid_v.at[b]], dc_v.at[b], gsem[b])

        def wait_g(b):
            pltpu.make_async_copy(qq_t.at[iq_v.at[b]], tq_v.at[b], gsem[b]).wait()
            pltpu.make_async_copy(rr_t.at[iqr_v.at[b]], tr_v.at[b], gsem[b]).wait()
            pltpu.make_async_copy(df_t.at[ipid_v.at[b]], dc_v.at[b], gsem[b]).wait()

        def compute(b):
            @pl.loop(0, _CHUNK // _LANES)
            def _(g):
                dg = dc_v[b, pl.ds(g * _LANES, _LANES)]
                acc_v[...] += dg * dg

            @pl.loop(0, _CHUNK)
            def _(i):
                dv = plsc.load_gather(
                    dc_v.at[b], [jnp.full((_LANES,), i, jnp.int32)])
                for cc in range(nchunk):
                    sl = pl.ds(cc * _LANES, _LANES)
                    sh = pl.ds(d + cc * _LANES, _LANES)
                    xo_v[i, sl] = tq_v[b, i, sl] + dv * tq_v[b, i, sh]
                    yo_v[i, sl] = tr_v[b, i, sl] + dv * tr_v[b, i, sh]

        def store(step):
            base = wid * per_w + step * _CHUNK
            pltpu.async_copy(xo_v, x_hbm.at[pl.ds(base, _CHUNK)], ssem)
            pltpu.async_copy(yo_v, y_hbm.at[pl.ds(base, _CHUNK)], ssem)

        def wait_s():
            pltpu.make_async_copy(xo_v, x_hbm.at[pl.ds(0, _CHUNK)], ssem).wait()
            pltpu.make_async_copy(yo_v, y_hbm.at[pl.ds(0, _CHUNK)], ssem).wait()

        fire(0, 0)

        @pl.loop(0, steps // 2)
        def _(p):
            s0 = 2 * p
            s1 = s0 + 1
            wait_g(0)
            fire(1, s1)
            compute(0)
            store(s0)
            wait_g(1)

            @pl.when(s1 + 1 < steps)
            def _():
                fire(0, s1 + 1)

            wait_s()
            compute(1)
            store(s1)
            wait_s()

        pltpu.sync_copy(acc_v, part_hbm.at[wid])

    return k(qf, qrf, pidf, qq, rr, dflat)


def _loss_tc(partials):
    def body(p_ref, o_ref):
        o_ref[0, 0] = jnp.sum(p_ref[...]) * jnp.float32(_L2)

    return pl.pallas_call(
        body,
        out_shape=jax.ShapeDtypeStruct((1, 1), jnp.float32),
        out_specs=pl.BlockSpec(memory_space=pltpu.SMEM),
    )(partials)


def kernel(q, qr, pid, q_emb, q_emb_diff, qr_emb, qr_emb_diff, diff_emb):
    b, l = q.shape
    d = q_emb.shape[1]
    qf = q.reshape(-1).astype(jnp.int32)
    qrf = qr.reshape(-1).astype(jnp.int32)
    pidf = pid.reshape(-1).astype(jnp.int32)
    qq = jnp.concatenate([q_emb, q_emb_diff], axis=1)
    rr = jnp.concatenate([qr_emb, qr_emb_diff], axis=1)
    dflat = diff_emb.reshape(-1)
    x, y, parts = _combine_sc(qf, qrf, pidf, qq, rr, dflat)
    loss = _loss_tc(parts)[0, 0]
    return x.reshape(b, l, d), y.reshape(b, l, d), loss
